# Initial kernel scaffold; baseline (speedup 1.0000x reference)
import jax
import jax.numpy as jnp
from jax.experimental import pallas as pl


def kernel(x, edge_index, batch, W1, b1, W2, b2, W3, b3, W4, b4, Wl, bl):
    raise NotImplementedError("write your pallas kernel here")


# speedup vs baseline: 9.2732x; 9.2732x over previous
import functools

import jax
import jax.numpy as jnp
from jax import lax
from jax.experimental import pallas as pl
from jax.experimental.pallas import tpu as pltpu
from jax.experimental.pallas import tpu_sc as plsc

_N = 10000
_E = 320000
_D = 128
_G = 64
_C = 10

_NP = 10240
_RPS = _NP // 16
_NC = 2
_NS = 16
_NW = _NC * _NS
_CH = 128
_E2 = _E + _N
_NCH = -(-_E2 // (_NW * _CH))
_EPT = _NCH * _CH
_EP = _EPT * _NW
_BR = 256
_NB = _NP // _BR

_mesh = plsc.VectorSubcoreMesh(core_axis_name="c", subcore_axis_name="s")


def _zero_fill(ref, nrows, ncols16):
  zvec = jnp.zeros((16,), jnp.float32)

  def row(r, carry):
    for j in range(ncols16):
      ref[r, pl.ds(j * 16, 16)] = zvec
    return carry

  lax.fori_loop(0, nrows, row, 0)


def _deg_body(dst_hbm, out_hbm, di, ones_b, zb, dacc, sem):
  c = lax.axis_index("c")
  s = lax.axis_index("s")
  wid = c * _NS + s

  ovec = jnp.full((16,), 1.0, jnp.float32)

  def orow(r, carry):
    ones_b[r, pl.ds(0, 16)] = ovec
    return carry

  lax.fori_loop(0, _CH, orow, 0)
  _zero_fill(zb, 64, 1)

  def zcp(k, carry):
    pltpu.sync_copy(zb, dacc.at[pl.ds(s * _RPS + k * 64, 64)])
    return carry

  lax.fori_loop(0, _RPS // 64, zcp, 0)
  plsc.subcore_barrier()

  ebase = wid * _EPT

  def body(i, carry):
    pltpu.sync_copy(dst_hbm.at[pl.ds(ebase + i * _CH, _CH)], di)
    pltpu.sync_copy(ones_b, dacc.at[di], add=True)
    return carry

  lax.fori_loop(0, _NCH, body, 0)
  plsc.subcore_barrier()

  pltpu.sync_copy(dacc.at[pl.ds(s * _RPS, _RPS)],
                  out_hbm.at[c, pl.ds(s * _RPS, _RPS)])


_deg_call = pl.kernel(
    _deg_body,
    out_type=jax.ShapeDtypeStruct((_NC, _NP, 16), jnp.float32),
    mesh=_mesh,
    scratch_types=[
        pltpu.VMEM((_CH,), jnp.int32),
        pltpu.VMEM((_CH, 16), jnp.float32),
        pltpu.VMEM((64, 16), jnp.float32),
        pltpu.VMEM_SHARED((_NP, 16), jnp.float32),
        pltpu.SemaphoreType.DMA,
    ],
)


def _agg_body(hp_hbm, src_hbm, dst_hbm, out_hbm, si, di, rows, zb, acc, sem):
  c = lax.axis_index("c")
  s = lax.axis_index("s")
  wid = c * _NS + s

  _zero_fill(zb, 64, 8)

  def zcp(k, carry):
    pltpu.sync_copy(zb, acc.at[pl.ds(s * _RPS + k * 64, 64)])
    return carry

  lax.fori_loop(0, _RPS // 64, zcp, 0)
  plsc.subcore_barrier()

  ebase = wid * _EPT

  def body(i, carry):
    b = ebase + i * _CH
    pltpu.sync_copy(src_hbm.at[pl.ds(b, _CH)], si)
    pltpu.sync_copy(dst_hbm.at[pl.ds(b, _CH)], di)
    pltpu.async_copy(hp_hbm.at[si], rows, sem).wait()
    pltpu.sync_copy(rows, acc.at[di], add=True)
    return carry

  lax.fori_loop(0, _NCH, body, 0)
  plsc.subcore_barrier()

  pltpu.sync_copy(acc.at[pl.ds(s * _RPS, _RPS)],
                  out_hbm.at[c, pl.ds(s * _RPS, _RPS)])


_agg_call = pl.kernel(
    _agg_body,
    out_type=jax.ShapeDtypeStruct((_NC, _NP, _D), jnp.float32),
    mesh=_mesh,
    scratch_types=[
        pltpu.VMEM((_CH,), jnp.int32),
        pltpu.VMEM((_CH,), jnp.int32),
        pltpu.VMEM((_CH, _D), jnp.float32),
        pltpu.VMEM((64, _D), jnp.float32),
        pltpu.VMEM_SHARED((_NP, _D), jnp.float32),
        pltpu.SemaphoreType.DMA,
    ],
)

_PREC = lax.Precision.HIGHEST


def _tc1_body(x_ref, dg_ref, w_ref, hp_ref, dis_ref):
  i = pl.program_id(0)
  deg = dg_ref[0] + dg_ref[1]
  degc = deg[:, 0:1]
  rid = lax.broadcasted_iota(jnp.int32, (_BR, 1), 0) + i * _BR
  dis = jnp.where(rid < _N, lax.rsqrt(degc), 0.0)
  disb = jnp.broadcast_to(dis, (_BR, _D))
  h = jnp.dot(x_ref[...], w_ref[...], precision=_PREC,
              preferred_element_type=jnp.float32)
  dis_ref[...] = disb
  hp_ref[...] = h * disb


def _tc1(xp, degs, W1):
  return pl.pallas_call(
      _tc1_body,
      grid=(_NB,),
      in_specs=[
          pl.BlockSpec((_BR, _D), lambda i: (i, 0)),
          pl.BlockSpec((_NC, _BR, 16), lambda i: (0, i, 0)),
          pl.BlockSpec((_D, _D), lambda i: (0, 0)),
      ],
      out_specs=[
          pl.BlockSpec((_BR, _D), lambda i: (i, 0)),
          pl.BlockSpec((_BR, _D), lambda i: (i, 0)),
      ],
      out_shape=[
          jax.ShapeDtypeStruct((_NP, _D), jnp.float32),
          jax.ShapeDtypeStruct((_NP, _D), jnp.float32),
      ],
  )(xp, degs, W1)


def _tcmid_body(a_ref, dis_ref, b_ref, w_ref, hp_ref):
  dis = dis_ref[...]
  y = (a_ref[0] + a_ref[1]) * dis + b_ref[...]
  y = jnp.maximum(y, 0.0)
  h = jnp.dot(y, w_ref[...], precision=_PREC,
              preferred_element_type=jnp.float32)
  hp_ref[...] = h * dis


def _tcmid(agg, disb, bvec, W):
  return pl.pallas_call(
      _tcmid_body,
      grid=(_NB,),
      in_specs=[
          pl.BlockSpec((_NC, _BR, _D), lambda i: (0, i, 0)),
          pl.BlockSpec((_BR, _D), lambda i: (i, 0)),
          pl.BlockSpec((1, _D), lambda i: (0, 0)),
          pl.BlockSpec((_D, _D), lambda i: (0, 0)),
      ],
      out_specs=pl.BlockSpec((_BR, _D), lambda i: (i, 0)),
      out_shape=jax.ShapeDtypeStruct((_NP, _D), jnp.float32),
  )(agg, disb, bvec, W)


def _tc5_body(a_ref, dis_ref, b_ref, bt_ref, wl_ref, bl_ref, out_ref,
              pooled, counts):
  i = pl.program_id(0)
  y = (a_ref[0] + a_ref[1]) * dis_ref[...] + b_ref[...]
  bt = bt_ref[0]
  gid = lax.broadcasted_iota(jnp.int32, (_G, _BR), 0)
  oh = (gid == jnp.broadcast_to(bt, (_G, _BR))).astype(jnp.float32)
  p = jnp.dot(oh, y, precision=_PREC, preferred_element_type=jnp.float32)
  cnt = jnp.dot(oh, jnp.ones((_BR, _D), jnp.float32), precision=_PREC,
                preferred_element_type=jnp.float32)

  @pl.when(i == 0)
  def _():
    pooled[...] = p
    counts[...] = cnt

  @pl.when(i > 0)
  def _():
    pooled[...] += p
    counts[...] += cnt

  @pl.when(i == _NB - 1)
  def _():
    mean = pooled[...] / jnp.maximum(counts[...], 1.0)
    out_ref[...] = jnp.dot(mean, wl_ref[...], precision=_PREC,
                           preferred_element_type=jnp.float32) + bl_ref[...]


def _tc5(agg, disb, b4v, batch3, Wlp, blp):
  return pl.pallas_call(
      _tc5_body,
      grid=(_NB,),
      in_specs=[
          pl.BlockSpec((_NC, _BR, _D), lambda i: (0, i, 0)),
          pl.BlockSpec((_BR, _D), lambda i: (i, 0)),
          pl.BlockSpec((1, _D), lambda i: (0, 0)),
          pl.BlockSpec((1, 1, _BR), lambda i: (i, 0, 0)),
          pl.BlockSpec((_D, _D), lambda i: (0, 0)),
          pl.BlockSpec((1, _D), lambda i: (0, 0)),
      ],
      out_specs=pl.BlockSpec((_G, _D), lambda i: (0, 0)),
      out_shape=jax.ShapeDtypeStruct((_G, _D), jnp.float32),
      scratch_shapes=[
          pltpu.VMEM((_G, _D), jnp.float32),
          pltpu.VMEM((_G, _D), jnp.float32),
      ],
  )(agg, disb, b4v, batch3, Wlp, blp)


def kernel(x, edge_index, batch, W1, b1, W2, b2, W3, b3, W4, b4, Wl, bl):
  f32 = jnp.float32
  x = x.astype(f32)

  src = edge_index[0]
  dst = edge_index[1]
  loop = jnp.arange(_N, dtype=jnp.int32)
  padi = jnp.full((_EP - _E2,), _NP - 1, jnp.int32)
  srcp = jnp.concatenate([src.astype(jnp.int32), loop, padi])
  dstp = jnp.concatenate([dst.astype(jnp.int32), loop, padi])

  xp = jnp.pad(x, ((0, _NP - _N), (0, 0)))
  batch_p = jnp.pad(batch.astype(jnp.int32), (0, _NP - _N),
                    constant_values=_G).reshape(_NB, 1, _BR)
  b1v = b1.astype(f32).reshape(1, _D)
  b2v = b2.astype(f32).reshape(1, _D)
  b3v = b3.astype(f32).reshape(1, _D)
  b4v = b4.astype(f32).reshape(1, _D)
  Wlp = jnp.pad(Wl.astype(f32), ((0, 0), (0, _D - _C)))
  blp = jnp.pad(bl.astype(f32), (0, _D - _C)).reshape(1, _D)

  degs = _deg_call(dstp)
  hp, disb = _tc1(xp, degs, W1.astype(f32))

  agg = _agg_call(hp, srcp, dstp)
  hp = _tcmid(agg, disb, b1v, W2.astype(f32))
  agg = _agg_call(hp, srcp, dstp)
  hp = _tcmid(agg, disb, b2v, W3.astype(f32))
  agg = _agg_call(hp, srcp, dstp)
  hp = _tcmid(agg, disb, b3v, W4.astype(f32))
  agg = _agg_call(hp, srcp, dstp)

  outp = _tc5(agg, disb, b4v, batch_p, Wlp, blp)
  return outp[:, :_C]
